# baseline (speedup 1.0000x reference)
import dataclasses

import jax
import jax.numpy as jnp
from jax import lax
from jax.experimental import pallas as pl
from jax.experimental.pallas import tpu as pltpu
from jax.experimental.pallas import tpu_sc as plsc

_NUM_EXPERTS = 64
_TOP_K = 8
_ALPHA = 0.01

_SC_CORES = 2
_SC_SUBCORES = 16
_LANES = 16
_NW = _SC_CORES * _SC_SUBCORES


def _sc_hist_body(idx_hbm, out_hbm, idx_v, hist_v, cnt_v, sem):
    chunk = idx_v.shape[0]
    wid = lax.axis_index("s") * _SC_CORES + lax.axis_index("c")
    base = wid * chunk
    cp = pltpu.make_async_copy(idx_hbm.at[pl.ds(base, chunk)], idx_v, sem)
    cp.start()

    zeros = jnp.zeros((_LANES,), jnp.int32)

    @pl.loop(0, _LANES * _NUM_EXPERTS, step=_LANES)
    def _(j):
        hist_v[pl.ds(j, _LANES)] = zeros

    cp.wait()

    lane_base = lax.iota(jnp.int32, _LANES) * _NUM_EXPERTS
    ones = jnp.ones((_LANES,), jnp.int32)

    @pl.loop(0, chunk, step=_LANES)
    def _(i):
        idx = idx_v[pl.ds(i, _LANES)]
        plsc.addupdate_scatter(hist_v, [lane_base + idx], ones)

    for j in range(0, _NUM_EXPERTS, _LANES):
        acc = hist_v[pl.ds(j, _LANES)]
        for r in range(1, _LANES):
            acc = acc + hist_v[pl.ds(r * _NUM_EXPERTS + j, _LANES)]
        cnt_v[pl.ds(j, _LANES)] = acc

    pltpu.sync_copy(cnt_v, out_hbm.at[wid])


def _sc_bincount(idx_flat):
    n = idx_flat.shape[0]
    chunk = n // _NW
    mesh = plsc.VectorSubcoreMesh(
        core_axis_name="c", subcore_axis_name="s",
        num_cores=_SC_CORES, num_subcores=_SC_SUBCORES,
    )
    cp = pltpu.CompilerParams()
    if "needs_layout_passes" in pltpu.CompilerParams.__dataclass_fields__:
        cp = dataclasses.replace(cp, needs_layout_passes=False)
    kern = pl.kernel(
        _sc_hist_body,
        out_type=jax.ShapeDtypeStruct((_NW, _NUM_EXPERTS), jnp.int32),
        mesh=mesh,
        compiler_params=cp,
        scratch_types=[
            pltpu.VMEM((chunk,), jnp.int32),
            pltpu.VMEM((_LANES * _NUM_EXPERTS,), jnp.int32),
            pltpu.VMEM((_NUM_EXPERTS,), jnp.int32),
            pltpu.SemaphoreType.DMA,
        ],
    )
    return kern(idx_flat)


def _tc_softmax_body(x_ref, o_ref):
    i = pl.program_id(0)
    x = x_ref[...]
    m = jnp.max(x, axis=1, keepdims=True)
    e = jnp.exp(x - m)
    s = jnp.sum(e, axis=1, keepdims=True)
    p = e / s
    part = jnp.sum(p.reshape(-1, 8, _NUM_EXPERTS), axis=0)

    @pl.when(i == 0)
    def _():
        o_ref[...] = part

    @pl.when(i != 0)
    def _():
        o_ref[...] = o_ref[...] + part


def _tc_softmax_colsum(x2d, blk):
    rows = x2d.shape[0]
    grid = rows // blk
    return pl.pallas_call(
        _tc_softmax_body,
        grid=(grid,),
        in_specs=[pl.BlockSpec((blk, _NUM_EXPERTS), lambda i: (i, 0))],
        out_specs=pl.BlockSpec((8, _NUM_EXPERTS), lambda i: (0, 0)),
        out_shape=jax.ShapeDtypeStruct((8, _NUM_EXPERTS), jnp.float32),
    )(x2d)


@jax.jit
def kernel(router_logits, expert_indices):
    batch, seq, _ = router_logits.shape
    num_tokens = batch * seq
    x2d = router_logits.reshape(num_tokens, _NUM_EXPERTS)
    idx_flat = expert_indices.reshape(-1)

    partial_counts = _sc_bincount(idx_flat)
    p_part = _tc_softmax_colsum(x2d, blk=4096)

    counts = jnp.sum(partial_counts, axis=0).astype(jnp.float32)
    p_sum = jnp.sum(p_part, axis=0)
    f_i = counts / (num_tokens * _TOP_K / _NUM_EXPERTS)
    p_i = p_sum / num_tokens
    return _ALPHA * jnp.sum(f_i * p_i) * _NUM_EXPERTS


# speedup vs baseline: 1.0007x; 1.0007x over previous
import dataclasses

import jax
import jax.numpy as jnp
from jax import lax
from jax.experimental import pallas as pl
from jax.experimental.pallas import tpu as pltpu
from jax.experimental.pallas import tpu_sc as plsc

_NUM_EXPERTS = 64
_TOP_K = 8
_ALPHA = 0.01

_SC_CORES = 2
_SC_SUBCORES = 16
_LANES = 16
_NW = _SC_CORES * _SC_SUBCORES


def _sc_hist_body(idx_hbm, out_hbm, idx_v, hist_v, cnt_v, sem):
    rows = idx_v.shape[0]
    wid = lax.axis_index("s") * _SC_CORES + lax.axis_index("c")
    base = wid * rows
    cp = pltpu.make_async_copy(idx_hbm.at[pl.ds(base, rows)], idx_v, sem)
    cp.start()

    zeros = jnp.zeros((_LANES,), jnp.int32)

    @pl.loop(0, _LANES * _NUM_EXPERTS, step=_LANES)
    def _(j):
        hist_v[pl.ds(j, _LANES)] = zeros

    cp.wait()

    lane_base = lax.iota(jnp.int32, _LANES) * _NUM_EXPERTS
    ones = jnp.ones((_LANES,), jnp.int32)

    @pl.loop(0, rows)
    def _(r):
        @pl.loop(0, 128, step=_LANES)
        def _(i):
            idx = idx_v[r, pl.ds(i, _LANES)]
            plsc.addupdate_scatter(hist_v, [lane_base + idx], ones)

    for j in range(0, _NUM_EXPERTS, _LANES):
        acc = hist_v[pl.ds(j, _LANES)]
        for r in range(1, _LANES):
            acc = acc + hist_v[pl.ds(r * _NUM_EXPERTS + j, _LANES)]
        cnt_v[pl.ds(j, _LANES)] = acc

    pltpu.sync_copy(cnt_v, out_hbm.at[wid])


def _sc_bincount(idx2d):
    rows_per_w = idx2d.shape[0] // _NW
    mesh = plsc.VectorSubcoreMesh(
        core_axis_name="c", subcore_axis_name="s",
        num_cores=_SC_CORES, num_subcores=_SC_SUBCORES,
    )
    cp = pltpu.CompilerParams()
    if "needs_layout_passes" in pltpu.CompilerParams.__dataclass_fields__:
        cp = dataclasses.replace(cp, needs_layout_passes=False)
    if "use_tc_tiling_on_sc" in pltpu.CompilerParams.__dataclass_fields__:
        cp = dataclasses.replace(cp, use_tc_tiling_on_sc=True)
    kern = pl.kernel(
        _sc_hist_body,
        out_type=jax.ShapeDtypeStruct((_NW, _NUM_EXPERTS), jnp.int32),
        mesh=mesh,
        compiler_params=cp,
        scratch_types=[
            pltpu.VMEM((rows_per_w, 128), jnp.int32),
            pltpu.VMEM((_LANES * _NUM_EXPERTS,), jnp.int32),
            pltpu.VMEM((_NUM_EXPERTS,), jnp.int32),
            pltpu.SemaphoreType.DMA,
        ],
    )
    return kern(idx2d)


def _tc_softmax_body(x_ref, o_ref):
    i = pl.program_id(0)
    x = x_ref[...]
    m = jnp.max(x, axis=1, keepdims=True)
    e = jnp.exp(x - m)
    s = jnp.sum(e, axis=1, keepdims=True)
    p = e / s
    part = jnp.sum(p.reshape(-1, 8, _NUM_EXPERTS), axis=0)

    @pl.when(i == 0)
    def _():
        o_ref[...] = part

    @pl.when(i != 0)
    def _():
        o_ref[...] = o_ref[...] + part


def _tc_softmax_colsum(x2d, blk):
    rows = x2d.shape[0]
    grid = rows // blk
    return pl.pallas_call(
        _tc_softmax_body,
        grid=(grid,),
        in_specs=[pl.BlockSpec((blk, _NUM_EXPERTS), lambda i: (i, 0))],
        out_specs=pl.BlockSpec((8, _NUM_EXPERTS), lambda i: (0, 0)),
        out_shape=jax.ShapeDtypeStruct((8, _NUM_EXPERTS), jnp.float32),
    )(x2d)


@jax.jit
def kernel(router_logits, expert_indices):
    batch, seq, _ = router_logits.shape
    num_tokens = batch * seq
    x2d = router_logits.reshape(num_tokens, _NUM_EXPERTS)
    idx2d = expert_indices.reshape(-1, 128)

    partial_counts = _sc_bincount(idx2d)
    p_part = _tc_softmax_colsum(x2d, blk=4096)

    counts = jnp.sum(partial_counts, axis=0).astype(jnp.float32)
    p_sum = jnp.sum(p_part, axis=0)
    f_i = counts / (num_tokens * _TOP_K / _NUM_EXPERTS)
    p_i = p_sum / num_tokens
    return _ALPHA * jnp.sum(f_i * p_i) * _NUM_EXPERTS


# speedup vs baseline: 1.8832x; 1.8820x over previous
import dataclasses

import jax
import jax.numpy as jnp
from jax import lax
from jax.experimental import pallas as pl
from jax.experimental.pallas import tpu as pltpu
from jax.experimental.pallas import tpu_sc as plsc

_NUM_EXPERTS = 64
_TOP_K = 8
_ALPHA = 0.01

_SC_CORES = 2
_SC_SUBCORES = 16
_LANES = 16
_NW = _SC_CORES * _SC_SUBCORES


def _sc_hist_body(idx_hbm, out_hbm, idx_v, hist_v, cnt_v, sem):
    n = idx_v.shape[0]
    wid = lax.axis_index("s") * _SC_CORES + lax.axis_index("c")
    cp = pltpu.make_async_copy(idx_hbm.at[wid], idx_v, sem)
    cp.start()

    zeros = jnp.zeros((_LANES,), jnp.int32)

    @pl.loop(0, _LANES * _NUM_EXPERTS, step=_LANES)
    def _(j):
        hist_v[pl.ds(j, _LANES)] = zeros

    cp.wait()

    lane_base = lax.iota(jnp.int32, _LANES) * _NUM_EXPERTS
    ones = jnp.ones((_LANES,), jnp.int32)

    @pl.loop(0, n, step=_LANES)
    def _(i):
        idx = idx_v[pl.ds(i, _LANES)]
        plsc.addupdate_scatter(hist_v, [lane_base + idx], ones)

    for j in range(0, _NUM_EXPERTS, _LANES):
        acc = hist_v[pl.ds(j, _LANES)]
        for r in range(1, _LANES):
            acc = acc + hist_v[pl.ds(r * _NUM_EXPERTS + j, _LANES)]
        cnt_v[pl.ds(j, _LANES)] = acc

    pltpu.sync_copy(cnt_v, out_hbm.at[wid])


def _sc_bincount(idx_rows):
    n = idx_rows.shape[1]
    mesh = plsc.VectorSubcoreMesh(
        core_axis_name="c", subcore_axis_name="s",
        num_cores=_SC_CORES, num_subcores=_SC_SUBCORES,
    )
    cp = pltpu.CompilerParams()
    if "needs_layout_passes" in pltpu.CompilerParams.__dataclass_fields__:
        cp = dataclasses.replace(cp, needs_layout_passes=False)
    kern = pl.kernel(
        _sc_hist_body,
        out_type=jax.ShapeDtypeStruct((_NW, _NUM_EXPERTS), jnp.int32),
        mesh=mesh,
        compiler_params=cp,
        scratch_types=[
            pltpu.VMEM((n,), jnp.int32),
            pltpu.VMEM((_LANES * _NUM_EXPERTS,), jnp.int32),
            pltpu.VMEM((_NUM_EXPERTS,), jnp.int32),
            pltpu.SemaphoreType.DMA,
        ],
    )
    return kern(idx_rows)


def _tc_softmax_body(x_ref, o_ref):
    j = pl.program_id(1)
    x = x_ref[0]
    m = jnp.max(x, axis=0, keepdims=True)
    e = jnp.exp(x - m)
    s = jnp.sum(e, axis=0, keepdims=True)
    p = e * (1.0 / s)
    part = p[:, 0:128]
    for c in range(1, p.shape[1] // 128):
        part = part + p[:, c * 128:(c + 1) * 128]

    @pl.when(j == 0)
    def _():
        o_ref[0] = part

    @pl.when(j != 0)
    def _():
        o_ref[0] = o_ref[0] + part


def _tc_softmax_rowsum(xt, seq_blk):
    batch, ne, seq = xt.shape
    grid = (batch, seq // seq_blk)
    acc = pl.pallas_call(
        _tc_softmax_body,
        grid=grid,
        in_specs=[pl.BlockSpec((1, ne, seq_blk), lambda b, j: (b, 0, j))],
        out_specs=pl.BlockSpec((1, ne, 128), lambda b, j: (b, 0, 0)),
        out_shape=jax.ShapeDtypeStruct((batch, ne, 128), jnp.float32),
        compiler_params=pltpu.CompilerParams(
            dimension_semantics=("parallel", "arbitrary"),
        ),
    )(xt)
    return acc


@jax.jit
def kernel(router_logits, expert_indices):
    batch, seq, _ = router_logits.shape
    num_tokens = batch * seq
    xt = jnp.transpose(router_logits, (0, 2, 1))
    idx_rows = jnp.transpose(expert_indices, (0, 2, 1)).reshape(_NW, -1)

    partial_counts = _sc_bincount(idx_rows)
    acc = _tc_softmax_rowsum(xt, seq_blk=2048)

    counts = jnp.sum(partial_counts, axis=0).astype(jnp.float32)
    p_sum = jnp.sum(acc, axis=(0, 2))
    f_i = counts / (num_tokens * _TOP_K / _NUM_EXPERTS)
    p_i = p_sum / num_tokens
    return _ALPHA * jnp.sum(f_i * p_i) * _NUM_EXPERTS
